# baseline (speedup 1.0000x reference)
import functools

import jax
import jax.numpy as jnp
from jax import lax
from jax.experimental import pallas as pl
from jax.experimental.pallas import tpu as pltpu
from jax.experimental.pallas import tpu_sc as plsc

NC = 2
NS = 16
NW = NC * NS

GROUP = 128
G_PER_CHUNK = 4
CHUNK = GROUP * G_PER_CHUNK


def _gather_kernel(n_groups_per_w, D, table_hbm, idx_hbm, out_hbm,
                   idx_v, rows_v, gsem, osem):
    wid = lax.axis_index("s") * NC + lax.axis_index("c")
    gbase = wid * n_groups_per_w
    pltpu.sync_copy(idx_hbm.at[pl.ds(gbase, n_groups_per_w)], idx_v)

    n_chunks = n_groups_per_w // G_PER_CHUNK

    def step(i, carry):
        copies = []
        for j in range(G_PER_CHUNK):
            g = i * G_PER_CHUNK + j
            copies.append(pltpu.async_copy(
                table_hbm.at[idx_v.at[g]],
                rows_v.at[pl.ds(j * GROUP, GROUP)],
                gsem))
        for c in copies:
            c.wait()
        row0 = (gbase * GROUP) + i * CHUNK
        pltpu.sync_copy(rows_v, out_hbm.at[pl.ds(row0, CHUNK)])
        return carry

    lax.fori_loop(0, n_chunks, step, 0)


def _make_gather(N, V, D):
    n_groups = N // GROUP
    n_groups_per_w = n_groups // NW
    mesh = plsc.VectorSubcoreMesh(core_axis_name="c", subcore_axis_name="s")
    return pl.kernel(
        functools.partial(_gather_kernel, n_groups_per_w, D),
        out_type=jax.ShapeDtypeStruct((N, D), jnp.float32),
        mesh=mesh,
        scratch_types=[
            pltpu.VMEM((n_groups_per_w, GROUP), jnp.int32),
            pltpu.VMEM((CHUNK, D), jnp.float32),
            pltpu.SemaphoreType.DMA,
            pltpu.SemaphoreType.DMA,
        ],
        compiler_params=pltpu.CompilerParams(use_tc_tiling_on_sc=False),
    )


def kernel(input_variable, weight):
    B, H = input_variable.shape
    V, D = weight.shape
    N = B * H
    idx = input_variable.reshape(N // GROUP, GROUP).astype(jnp.int32)
    out = _make_gather(N, V, D)(weight, idx)
    return out.reshape(B, H, D)


# speedup vs baseline: 1.0219x; 1.0219x over previous
import functools

import jax
import jax.numpy as jnp
from jax import lax
from jax.experimental import pallas as pl
from jax.experimental.pallas import tpu as pltpu
from jax.experimental.pallas import tpu_sc as plsc

NC = 2
NS = 16
NW = NC * NS

GROUP = 128
G_PER_CHUNK = 4
CHUNK = GROUP * G_PER_CHUNK
NBUF = 3


def _gather_kernel(n_groups_per_w, D, table_hbm, idx_hbm, out_hbm,
                   idx_v, rows_v, gsem, osem):
    wid = lax.axis_index("s") * NC + lax.axis_index("c")
    gbase = wid * n_groups_per_w
    out0 = gbase * GROUP
    pltpu.sync_copy(idx_hbm.at[pl.ds(gbase, n_groups_per_w)], idx_v)

    n_chunks = n_groups_per_w // G_PER_CHUNK

    def fire(c, b):
        for j in range(G_PER_CHUNK):
            pltpu.async_copy(
                table_hbm.at[idx_v.at[c * G_PER_CHUNK + j]],
                rows_v.at[b, pl.ds(j * GROUP, GROUP)],
                gsem.at[b])

    def drain_gathers(b):
        for j in range(G_PER_CHUNK):
            pltpu.make_async_copy(
                table_hbm.at[idx_v.at[j]],
                rows_v.at[b, pl.ds(j * GROUP, GROUP)],
                gsem.at[b]).wait()

    def start_writeback(c, b):
        pltpu.async_copy(
            rows_v.at[b],
            out_hbm.at[pl.ds(out0 + c * CHUNK, CHUNK)],
            osem.at[b])

    def drain_writeback(b):
        pltpu.make_async_copy(
            rows_v.at[b],
            out_hbm.at[pl.ds(out0, CHUNK)],
            osem.at[b]).wait()

    fire(0, 0)
    fire(1, 1)

    def step(c, carry):
        b = lax.rem(c, NBUF)
        nb = lax.rem(c + 2, NBUF)

        @pl.when(c + 2 < n_chunks)
        def _():
            @pl.when(c >= 1)
            def _():
                drain_writeback(nb)
            fire(c + 2, nb)

        drain_gathers(b)
        start_writeback(c, b)
        return carry

    lax.fori_loop(0, n_chunks, step, 0)
    for tail in range(NBUF):
        drain_writeback((n_chunks - NBUF + tail) % NBUF)


def _make_gather(N, V, D):
    n_groups = N // GROUP
    n_groups_per_w = n_groups // NW
    mesh = plsc.VectorSubcoreMesh(core_axis_name="c", subcore_axis_name="s")
    return pl.kernel(
        functools.partial(_gather_kernel, n_groups_per_w, D),
        out_type=jax.ShapeDtypeStruct((N, D), jnp.float32),
        mesh=mesh,
        scratch_types=[
            pltpu.VMEM((n_groups_per_w, GROUP), jnp.int32),
            pltpu.VMEM((NBUF, CHUNK, D), jnp.float32),
            pltpu.SemaphoreType.DMA((NBUF,)),
            pltpu.SemaphoreType.DMA((NBUF,)),
        ],
        compiler_params=pltpu.CompilerParams(use_tc_tiling_on_sc=False),
    )


def kernel(input_variable, weight):
    B, H = input_variable.shape
    V, D = weight.shape
    N = B * H
    idx = input_variable.reshape(N // GROUP, GROUP).astype(jnp.int32)
    out = _make_gather(N, V, D)(weight, idx)
    return out.reshape(B, H, D)
